# baseline (speedup 1.0000x reference)
import functools

import jax
import jax.numpy as jnp
from jax import lax
from jax.experimental import pallas as pl
from jax.experimental.pallas import tpu as pltpu
from jax.experimental.pallas import tpu_sc as plsc

BATCH = 16384
MF = 200
MF_PAD = 208
DMLP = 32
NC = 2
NS = 16
NW = NC * NS
ROWS_PER_W = BATCH // NW
CHUNK = 128
NCHUNK = ROWS_PER_W // CHUNK
L = 16


def _sc_gather_reduce(userinput, iteminput, mf_user_w, mlp_user_w,
                      mlp_item_w, wp_mf_pad):
    mesh = plsc.VectorSubcoreMesh(core_axis_name="c", subcore_axis_name="s")

    @functools.partial(
        pl.kernel,
        mesh=mesh,
        compiler_params=pltpu.CompilerParams(
            needs_layout_passes=False, use_tc_tiling_on_sc=False),
        out_type=(
            jax.ShapeDtypeStruct((BATCH, DMLP), jnp.float32),
            jax.ShapeDtypeStruct((BATCH, DMLP), jnp.float32),
            jax.ShapeDtypeStruct((BATCH,), jnp.float32),
        ),
        scratch_types=[
            pltpu.VMEM((CHUNK,), jnp.int32),
            pltpu.VMEM((CHUNK,), jnp.int32),
            pltpu.VMEM((CHUNK, MF), jnp.float32),
            pltpu.VMEM((CHUNK, DMLP), jnp.float32),
            pltpu.VMEM((CHUNK, DMLP), jnp.float32),
            pltpu.VMEM((MF_PAD,), jnp.float32),
            pltpu.VMEM((CHUNK,), jnp.float32),
            pltpu.SemaphoreType.DMA,
            pltpu.SemaphoreType.DMA,
            pltpu.SemaphoreType.DMA,
        ],
    )
    def k(u_hbm, i_hbm, mfw_hbm, mlpu_hbm, mlpi_hbm, wp_hbm,
          out_u, out_i, out_mf,
          uidx, iidx, mfrows, urows, irows, wpv, mfo, sem0, sem1, sem2):
        wid = lax.axis_index("s") * NC + lax.axis_index("c")
        pltpu.sync_copy(wp_hbm, wpv)
        wps = [wpv[pl.ds(16 * j, 16)] for j in range(MF_PAD // 16)]
        lane = lax.broadcasted_iota(jnp.int32, (L,), 0)
        for c in range(NCHUNK):
            rb = pl.multiple_of(wid * ROWS_PER_W + c * CHUNK, CHUNK)
            pltpu.sync_copy(u_hbm.at[pl.ds(rb, CHUNK)], uidx)
            pltpu.sync_copy(i_hbm.at[pl.ds(rb, CHUNK)], iidx)
            cp0 = pltpu.async_copy(mfw_hbm.at[uidx], mfrows, sem0)
            cp1 = pltpu.async_copy(mlpu_hbm.at[uidx], urows, sem1)
            cp2 = pltpu.async_copy(mlpi_hbm.at[iidx], irows, sem2)
            cp1.wait()
            pltpu.sync_copy(urows, out_u.at[pl.ds(rb, CHUNK)])
            cp2.wait()
            pltpu.sync_copy(irows, out_i.at[pl.ds(rb, CHUNK)])
            cp0.wait()

            def group(g, carry):
                vec = jnp.zeros((L,), jnp.float32)
                for kk in range(L):
                    r = g * L + kk
                    acc = None
                    for j in range(MF_PAD // 16):
                        src = 16 * j if j < 12 else MF - 16
                        v = mfrows[r, pl.ds(src, 16)]
                        t = (v * v) * wps[j]
                        acc = t if acc is None else acc + t
                    s = jnp.sum(acc)
                    vec = jnp.where(lane == kk, s, vec)
                mfo[pl.ds(g * L, L)] = vec
                return carry

            lax.fori_loop(0, CHUNK // L, group, 0)
            pltpu.sync_copy(mfo, out_mf.at[pl.ds(rb, CHUNK)])

    return k(userinput, iteminput, mf_user_w, mlp_user_w, mlp_item_w,
             wp_mf_pad)


def _tc_head(u_rows, i_rows, mf_part, w1ut, w1it, b1, w2t, b2, w3t, b3,
             wpm, bp):
    def body(u_ref, i_ref, mf_ref, w1u_ref, w1i_ref, b1_ref, w2_ref, b2_ref,
             w3_ref, b3_ref, wpm_ref, bp_ref, o_ref):
        h = jnp.dot(u_ref[...], w1u_ref[...],
                    preferred_element_type=jnp.float32)
        h = h + jnp.dot(i_ref[...], w1i_ref[...],
                        preferred_element_type=jnp.float32)
        h = jnp.maximum(h + b1_ref[...], 0.0)
        h = jnp.maximum(jnp.dot(h, w2_ref[...],
                                preferred_element_type=jnp.float32)
                        + b2_ref[...], 0.0)
        h = jnp.maximum(jnp.dot(h, w3_ref[...],
                                preferred_element_type=jnp.float32)
                        + b3_ref[...], 0.0)
        logit = jnp.sum(h * wpm_ref[...], axis=-1)
        logit = logit + mf_ref[...] + bp_ref[0, 0]
        o_ref[...] = 1.0 / (1.0 + jnp.exp(-logit))

    return pl.pallas_call(
        body,
        out_shape=jax.ShapeDtypeStruct((BATCH,), jnp.float32),
    )(u_rows, i_rows, mf_part, w1ut, w1it, b1, w2t, b2, w3t, b3, wpm, bp)


def kernel(userinput, iteminput, mf_user_w, mf_item_w, mlp_user_w,
           mlp_item_w, W1, b1, W2, b2, W3, b3, Wp, bp):
    del mf_item_w
    wp = Wp.reshape(-1)
    wp_mf_pad = jnp.concatenate(
        [wp[:192], jnp.zeros((8,), jnp.float32), wp[192:200]])
    u_rows, i_rows, mf_part = _sc_gather_reduce(
        userinput, iteminput, mf_user_w, mlp_user_w, mlp_item_w, wp_mf_pad)
    out = _tc_head(
        u_rows, i_rows, mf_part,
        W1[:, :DMLP].T, W1[:, DMLP:].T, b1.reshape(1, -1),
        W2.T, b2.reshape(1, -1), W3.T, b3.reshape(1, -1),
        wp[200:].reshape(1, -1), bp.reshape(1, 1))
    return out


# speedup vs baseline: 2.0451x; 2.0451x over previous
import functools

import jax
import jax.numpy as jnp
from jax import lax
from jax.experimental import pallas as pl
from jax.experimental.pallas import tpu as pltpu
from jax.experimental.pallas import tpu_sc as plsc

BATCH = 16384
NUSERS = 100000
MF = 200
DMLP = 32
NC = 2
NS = 16
NW = NC * NS
ROWS_PER_W = BATCH // NW
CHUNK = 128
NCHUNK = ROWS_PER_W // CHUNK
SBLK = 1024


def _tc_mf_reduce(mf_user_w, wp_mf):
    def body(t_ref, wp_ref, s_ref):
        x = t_ref[...]
        s_ref[...] = jnp.sum(x * x * wp_ref[...], axis=1)

    return pl.pallas_call(
        body,
        grid=(pl.cdiv(NUSERS, SBLK),),
        in_specs=[
            pl.BlockSpec((SBLK, MF), lambda i: (i, 0)),
            pl.BlockSpec((1, MF), lambda i: (0, 0)),
        ],
        out_specs=pl.BlockSpec((SBLK,), lambda i: (i,)),
        out_shape=jax.ShapeDtypeStruct((NUSERS,), jnp.float32),
    )(mf_user_w, wp_mf)


def _sc_gather(userinput, iteminput, s_table, mlp_user_w, mlp_item_w):
    mesh = plsc.VectorSubcoreMesh(core_axis_name="c", subcore_axis_name="s")

    @functools.partial(
        pl.kernel,
        mesh=mesh,
        compiler_params=pltpu.CompilerParams(
            needs_layout_passes=False, use_tc_tiling_on_sc=False),
        out_type=(
            jax.ShapeDtypeStruct((BATCH, DMLP), jnp.float32),
            jax.ShapeDtypeStruct((BATCH, DMLP), jnp.float32),
            jax.ShapeDtypeStruct((BATCH,), jnp.float32),
        ),
        scratch_types=[
            pltpu.VMEM((CHUNK,), jnp.int32),
            pltpu.VMEM((CHUNK,), jnp.int32),
            pltpu.VMEM((CHUNK, DMLP), jnp.float32),
            pltpu.VMEM((CHUNK, DMLP), jnp.float32),
            pltpu.VMEM((CHUNK,), jnp.float32),
            pltpu.SemaphoreType.DMA,
            pltpu.SemaphoreType.DMA,
            pltpu.SemaphoreType.DMA,
        ],
    )
    def k(u_hbm, i_hbm, s_hbm, mlpu_hbm, mlpi_hbm,
          out_u, out_i, out_s,
          uidx, iidx, urows, irows, svals, sem0, sem1, sem2):
        wid = lax.axis_index("s") * NC + lax.axis_index("c")
        for c in range(NCHUNK):
            rb = pl.multiple_of(wid * ROWS_PER_W + c * CHUNK, CHUNK)
            pltpu.sync_copy(u_hbm.at[pl.ds(rb, CHUNK)], uidx)
            pltpu.sync_copy(i_hbm.at[pl.ds(rb, CHUNK)], iidx)
            cp0 = pltpu.async_copy(s_hbm.at[uidx], svals, sem0)
            cp1 = pltpu.async_copy(mlpu_hbm.at[uidx], urows, sem1)
            cp2 = pltpu.async_copy(mlpi_hbm.at[iidx], irows, sem2)
            cp1.wait()
            pltpu.sync_copy(urows, out_u.at[pl.ds(rb, CHUNK)])
            cp2.wait()
            pltpu.sync_copy(irows, out_i.at[pl.ds(rb, CHUNK)])
            cp0.wait()
            pltpu.sync_copy(svals, out_s.at[pl.ds(rb, CHUNK)])

    return k(userinput, iteminput, s_table, mlp_user_w, mlp_item_w)


def _tc_head(u_rows, i_rows, mf_part, w1ut, w1it, b1, w2t, b2, w3t, b3,
             wpm, bp):
    def body(u_ref, i_ref, mf_ref, w1u_ref, w1i_ref, b1_ref, w2_ref, b2_ref,
             w3_ref, b3_ref, wpm_ref, bp_ref, o_ref):
        h = jnp.dot(u_ref[...], w1u_ref[...],
                    preferred_element_type=jnp.float32)
        h = h + jnp.dot(i_ref[...], w1i_ref[...],
                        preferred_element_type=jnp.float32)
        h = jnp.maximum(h + b1_ref[...], 0.0)
        h = jnp.maximum(jnp.dot(h, w2_ref[...],
                                preferred_element_type=jnp.float32)
                        + b2_ref[...], 0.0)
        h = jnp.maximum(jnp.dot(h, w3_ref[...],
                                preferred_element_type=jnp.float32)
                        + b3_ref[...], 0.0)
        logit = jnp.sum(h * wpm_ref[...], axis=-1)
        logit = logit + mf_ref[...] + bp_ref[0, 0]
        o_ref[...] = 1.0 / (1.0 + jnp.exp(-logit))

    return pl.pallas_call(
        body,
        out_shape=jax.ShapeDtypeStruct((BATCH,), jnp.float32),
    )(u_rows, i_rows, mf_part, w1ut, w1it, b1, w2t, b2, w3t, b3, wpm, bp)


def kernel(userinput, iteminput, mf_user_w, mf_item_w, mlp_user_w,
           mlp_item_w, W1, b1, W2, b2, W3, b3, Wp, bp):
    del mf_item_w
    wp = Wp.reshape(-1)
    s_table = _tc_mf_reduce(mf_user_w, wp[:MF].reshape(1, MF))
    u_rows, i_rows, mf_part = _sc_gather(
        userinput, iteminput, s_table, mlp_user_w, mlp_item_w)
    out = _tc_head(
        u_rows, i_rows, mf_part,
        W1[:, :DMLP].T, W1[:, DMLP:].T, b1.reshape(1, -1),
        W2.T, b2.reshape(1, -1), W3.T, b3.reshape(1, -1),
        wp[MF:].reshape(1, -1), bp.reshape(1, 1))
    return out


# speedup vs baseline: 3.3083x; 1.6177x over previous
import functools

import jax
import jax.numpy as jnp
from jax import lax
from jax.experimental import pallas as pl
from jax.experimental.pallas import tpu as pltpu
from jax.experimental.pallas import tpu_sc as plsc

BATCH = 16384
NUSERS = 100000
MF = 200
DMLP = 32
NC = 2
NS = 16
NW = NC * NS
ROWS_PER_W = BATCH // NW
CHUNK = 128
NCHUNK = ROWS_PER_W // CHUNK
SBLK = 2048


def _tc_mf_reduce(mf_t, wp_col):
    def body(t_ref, wp_ref, s_ref):
        x = t_ref[...]
        s_ref[...] = jnp.sum(x * x * wp_ref[...], axis=0)

    return pl.pallas_call(
        body,
        grid=(pl.cdiv(NUSERS, SBLK),),
        in_specs=[
            pl.BlockSpec((MF, SBLK), lambda i: (0, i)),
            pl.BlockSpec((MF, 1), lambda i: (0, 0)),
        ],
        out_specs=pl.BlockSpec((SBLK,), lambda i: (i,)),
        out_shape=jax.ShapeDtypeStruct((NUSERS,), jnp.float32),
    )(mf_t, wp_col)


def _sc_gather(userinput, iteminput, s_table, mlp_user_w, mlp_item_w):
    mesh = plsc.VectorSubcoreMesh(core_axis_name="c", subcore_axis_name="s")

    @functools.partial(
        pl.kernel,
        mesh=mesh,
        compiler_params=pltpu.CompilerParams(
            needs_layout_passes=False, use_tc_tiling_on_sc=False),
        out_type=(
            jax.ShapeDtypeStruct((BATCH, DMLP), jnp.float32),
            jax.ShapeDtypeStruct((BATCH, DMLP), jnp.float32),
            jax.ShapeDtypeStruct((BATCH,), jnp.float32),
        ),
        scratch_types=[
            pltpu.VMEM((CHUNK,), jnp.int32),
            pltpu.VMEM((CHUNK,), jnp.int32),
            pltpu.VMEM((CHUNK, DMLP), jnp.float32),
            pltpu.VMEM((CHUNK, DMLP), jnp.float32),
            pltpu.VMEM((CHUNK,), jnp.float32),
            pltpu.SemaphoreType.DMA,
            pltpu.SemaphoreType.DMA,
            pltpu.SemaphoreType.DMA,
        ],
    )
    def k(u_hbm, i_hbm, s_hbm, mlpu_hbm, mlpi_hbm,
          out_u, out_i, out_s,
          uidx, iidx, urows, irows, svals, sem0, sem1, sem2):
        wid = lax.axis_index("s") * NC + lax.axis_index("c")
        for c in range(NCHUNK):
            rb = pl.multiple_of(wid * ROWS_PER_W + c * CHUNK, CHUNK)
            pltpu.sync_copy(u_hbm.at[pl.ds(rb, CHUNK)], uidx)
            pltpu.sync_copy(i_hbm.at[pl.ds(rb, CHUNK)], iidx)
            cp0 = pltpu.async_copy(s_hbm.at[uidx], svals, sem0)
            cp1 = pltpu.async_copy(mlpu_hbm.at[uidx], urows, sem1)
            cp2 = pltpu.async_copy(mlpi_hbm.at[iidx], irows, sem2)
            cp1.wait()
            pltpu.sync_copy(urows, out_u.at[pl.ds(rb, CHUNK)])
            cp2.wait()
            pltpu.sync_copy(irows, out_i.at[pl.ds(rb, CHUNK)])
            cp0.wait()
            pltpu.sync_copy(svals, out_s.at[pl.ds(rb, CHUNK)])

    return k(userinput, iteminput, s_table, mlp_user_w, mlp_item_w)


def _tc_head(u_rows, i_rows, mf_part, w1ut, w1it, b1, w2t, b2, w3t, b3,
             wpm, bp):
    def body(u_ref, i_ref, mf_ref, w1u_ref, w1i_ref, b1_ref, w2_ref, b2_ref,
             w3_ref, b3_ref, wpm_ref, bp_ref, o_ref):
        h = jnp.dot(u_ref[...], w1u_ref[...],
                    preferred_element_type=jnp.float32)
        h = h + jnp.dot(i_ref[...], w1i_ref[...],
                        preferred_element_type=jnp.float32)
        h = jnp.maximum(h + b1_ref[...], 0.0)
        h = jnp.maximum(jnp.dot(h, w2_ref[...],
                                preferred_element_type=jnp.float32)
                        + b2_ref[...], 0.0)
        h = jnp.maximum(jnp.dot(h, w3_ref[...],
                                preferred_element_type=jnp.float32)
                        + b3_ref[...], 0.0)
        logit = jnp.sum(h * wpm_ref[...], axis=-1)
        logit = logit + mf_ref[...] + bp_ref[0, 0]
        o_ref[...] = 1.0 / (1.0 + jnp.exp(-logit))

    return pl.pallas_call(
        body,
        out_shape=jax.ShapeDtypeStruct((BATCH,), jnp.float32),
    )(u_rows, i_rows, mf_part, w1ut, w1it, b1, w2t, b2, w3t, b3, wpm, bp)


def kernel(userinput, iteminput, mf_user_w, mf_item_w, mlp_user_w,
           mlp_item_w, W1, b1, W2, b2, W3, b3, Wp, bp):
    del mf_item_w
    wp = Wp.reshape(-1)
    s_table = _tc_mf_reduce(mf_user_w.T, wp[:MF].reshape(MF, 1))
    u_rows, i_rows, mf_part = _sc_gather(
        userinput, iteminput, s_table, mlp_user_w, mlp_item_w)
    out = _tc_head(
        u_rows, i_rows, mf_part,
        W1[:, :DMLP].T, W1[:, DMLP:].T, b1.reshape(1, -1),
        W2.T, b2.reshape(1, -1), W3.T, b3.reshape(1, -1),
        wp[MF:].reshape(1, -1), bp.reshape(1, 1))
    return out


# speedup vs baseline: 5.0724x; 1.5332x over previous
import functools

import jax
import jax.numpy as jnp
from jax import lax
from jax.experimental import pallas as pl
from jax.experimental.pallas import tpu as pltpu
from jax.experimental.pallas import tpu_sc as plsc

BATCH = 16384
NUSERS = 100000
MF = 200
DMLP = 32
PW = 128
NC = 2
NS = 16
NW = NC * NS
ROWS_PER_W = BATCH // NW
CHUNK = 128
NCHUNK = ROWS_PER_W // CHUNK
PBLK = 2048


def _tc_build(mf_t, mut, mit, wp_col):
    def body(mft_ref, mut_ref, mit_ref, wp_ref, p_ref, s_ref):
        x = mft_ref[...]
        s_ref[...] = jnp.sum(x * x * wp_ref[...], axis=0)
        p_ref[:, 0:DMLP] = jnp.transpose(mut_ref[...])
        p_ref[:, DMLP:2 * DMLP] = jnp.transpose(mit_ref[...])

    return pl.pallas_call(
        body,
        grid=(pl.cdiv(NUSERS, PBLK),),
        in_specs=[
            pl.BlockSpec((MF, PBLK), lambda i: (0, i)),
            pl.BlockSpec((DMLP, PBLK), lambda i: (0, i)),
            pl.BlockSpec((DMLP, PBLK), lambda i: (0, i)),
            pl.BlockSpec((MF, 1), lambda i: (0, 0)),
        ],
        out_specs=(pl.BlockSpec((PBLK, PW), lambda i: (i, 0)),
                   pl.BlockSpec((PBLK,), lambda i: (i,))),
        out_shape=(jax.ShapeDtypeStruct((NUSERS, PW), jnp.float32),
                   jax.ShapeDtypeStruct((NUSERS,), jnp.float32)),
    )(mf_t, mut, mit, wp_col)


def _sc_gather(userinput, iteminput, p_table, s_table):
    mesh = plsc.VectorSubcoreMesh(core_axis_name="c", subcore_axis_name="s")

    @functools.partial(
        pl.kernel,
        mesh=mesh,
        compiler_params=pltpu.CompilerParams(needs_layout_passes=False),
        out_type=(
            jax.ShapeDtypeStruct((BATCH, PW), jnp.float32),
            jax.ShapeDtypeStruct((BATCH,), jnp.float32),
        ),
        scratch_types=[
            pltpu.VMEM((CHUNK,), jnp.int32),
            pltpu.VMEM((CHUNK,), jnp.int32),
            pltpu.VMEM((CHUNK, PW), jnp.float32),
            pltpu.VMEM((CHUNK, PW), jnp.float32),
            pltpu.VMEM((CHUNK,), jnp.float32),
            pltpu.SemaphoreType.DMA,
            pltpu.SemaphoreType.DMA,
            pltpu.SemaphoreType.DMA,
        ],
    )
    def k(u_hbm, i_hbm, p_hbm, s_hbm,
          out_cat, out_s,
          uidx, iidx, urows, irows, svals, sem0, sem1, sem2):
        wid = lax.axis_index("s") * NC + lax.axis_index("c")
        for c in range(NCHUNK):
            rb = pl.multiple_of(wid * ROWS_PER_W + c * CHUNK, CHUNK)
            pltpu.sync_copy(u_hbm.at[pl.ds(rb, CHUNK)], uidx)
            pltpu.sync_copy(i_hbm.at[pl.ds(rb, CHUNK)], iidx)
            cp0 = pltpu.async_copy(s_hbm.at[uidx], svals, sem0)
            cp1 = pltpu.async_copy(p_hbm.at[uidx], urows, sem1)
            cp2 = pltpu.async_copy(p_hbm.at[iidx], irows, sem2)
            cp0.wait()
            pltpu.sync_copy(svals, out_s.at[pl.ds(rb, CHUNK)])
            cp1.wait()
            cp2.wait()

            def merge(r, carry):
                urows[r, pl.ds(DMLP, 16)] = irows[r, pl.ds(DMLP, 16)]
                urows[r, pl.ds(DMLP + 16, 16)] = irows[r, pl.ds(DMLP + 16, 16)]
                return carry

            lax.fori_loop(0, CHUNK, merge, 0)
            pltpu.sync_copy(urows, out_cat.at[pl.ds(rb, CHUNK)])

    return k(userinput, iteminput, p_table, s_table)


def _tc_head(x_cat, mf_part, w1t, b1, w2t, b2, w3t, b3, wpm, bp):
    def body(x_ref, mf_ref, w1_ref, b1_ref, w2_ref, b2_ref,
             w3_ref, b3_ref, wpm_ref, bp_ref, o_ref):
        h = jnp.dot(x_ref[:, 0:2 * DMLP], w1_ref[...],
                    preferred_element_type=jnp.float32)
        h = jnp.maximum(h + b1_ref[...], 0.0)
        h = jnp.maximum(jnp.dot(h, w2_ref[...],
                                preferred_element_type=jnp.float32)
                        + b2_ref[...], 0.0)
        h = jnp.maximum(jnp.dot(h, w3_ref[...],
                                preferred_element_type=jnp.float32)
                        + b3_ref[...], 0.0)
        logit = jnp.sum(h * wpm_ref[...], axis=-1)
        logit = logit + mf_ref[...] + bp_ref[0, 0]
        o_ref[...] = 1.0 / (1.0 + jnp.exp(-logit))

    return pl.pallas_call(
        body,
        out_shape=jax.ShapeDtypeStruct((BATCH,), jnp.float32),
    )(x_cat, mf_part, w1t, b1, w2t, b2, w3t, b3, wpm, bp)


def kernel(userinput, iteminput, mf_user_w, mf_item_w, mlp_user_w,
           mlp_item_w, W1, b1, W2, b2, W3, b3, Wp, bp):
    del mf_item_w
    wp = Wp.reshape(-1)
    p_table, s_table = _tc_build(
        mf_user_w.T, mlp_user_w.T, mlp_item_w.T, wp[:MF].reshape(MF, 1))
    x_cat, mf_part = _sc_gather(userinput, iteminput, p_table, s_table)
    out = _tc_head(
        x_cat, mf_part,
        W1.T, b1.reshape(1, -1),
        W2.T, b2.reshape(1, -1), W3.T, b3.reshape(1, -1),
        wp[MF:].reshape(1, -1), bp.reshape(1, 1))
    return out
